# baseline (speedup 1.0000x reference)
import functools
import math

import numpy as np
import jax
import jax.numpy as jnp
from jax import lax
from jax.experimental import pallas as pl
from jax.experimental.pallas import tpu as pltpu

B, L0, C_IN = 2, 2048, 7
D_MODEL, N_HEADS, E_LAYERS = 1024, 16, 3
D_FF, FACTOR = 512, 3
DK = D_MODEL // N_HEADS

PREC = lax.Precision.DEFAULT
F32 = jnp.float32

def _sample_constants(layer_idx, Ll):
    u = min(int(FACTOR * np.ceil(np.log(Ll))), Ll)
    rng = jax.random.fold_in(jax.random.key(42), layer_idx)
    idx = jax.random.randint(rng, (Ll, u), 0, Ll)
    cnt = jnp.zeros((Ll, Ll), F32)
    cnt = cnt.at[jnp.arange(Ll)[:, None], idx].add(1.0)
    return u, cnt.astype(jnp.bfloat16)


def _embed_body(x_ref, w0_ref, w1_ref, w2_ref, o_ref):
    x = x_ref[...]
    xm1 = jnp.concatenate([x[-1:, :], x[:-1, :]], axis=0)
    xp1 = jnp.concatenate([x[1:, :], x[:1, :]], axis=0)
    o_ref[...] = (
        jnp.dot(xm1, w0_ref[...], precision=PREC, preferred_element_type=F32)
        + jnp.dot(x, w1_ref[...], precision=PREC, preferred_element_type=F32)
        + jnp.dot(xp1, w2_ref[...], precision=PREC, preferred_element_type=F32)
    )


def _embed(x_flat, w):
    w0 = w[:, :, 0].T
    w1 = w[:, :, 1].T
    w2 = w[:, :, 2].T
    return pl.pallas_call(
        _embed_body,
        grid=(B,),
        in_specs=[
            pl.BlockSpec((L0, C_IN), lambda b: (b, 0)),
            pl.BlockSpec((C_IN, D_MODEL), lambda b: (0, 0)),
            pl.BlockSpec((C_IN, D_MODEL), lambda b: (0, 0)),
            pl.BlockSpec((C_IN, D_MODEL), lambda b: (0, 0)),
        ],
        out_specs=pl.BlockSpec((L0, D_MODEL), lambda b: (b, 0)),
        out_shape=jax.ShapeDtypeStruct((B * L0, D_MODEL), F32),
    )(x_flat, w0, w1, w2)


def _qkv_body(x_ref, w_ref, b_ref, o_ref):
    o_ref[...] = (
        jnp.dot(x_ref[...], w_ref[...], precision=PREC, preferred_element_type=F32)
        + b_ref[...]
    )


def _qkv(x_flat, lp, rows):
    w = jnp.concatenate([lp["Wq"], lp["Wk"], lp["Wv"]], axis=0).T
    b = jnp.concatenate([lp["bq"], lp["bk"], lp["bv"]])[None, :]
    TM = 256
    nt = rows // TM
    return pl.pallas_call(
        _qkv_body,
        grid=(nt,),
        in_specs=[
            pl.BlockSpec((TM, D_MODEL), lambda i: (i, 0)),
            pl.BlockSpec((D_MODEL, 3 * D_MODEL), lambda i: (0, 0)),
            pl.BlockSpec((1, 3 * D_MODEL), lambda i: (0, 0)),
        ],
        out_specs=pl.BlockSpec((TM, 3 * D_MODEL), lambda i: (i, 0)),
        out_shape=jax.ShapeDtypeStruct((rows, 3 * D_MODEL), F32),
    )(x_flat, w, b)


def _attn_one_head(q, k, v, c_ref, *, Lx, u):
    TS = 256
    ntile = Lx // TS

    cols = []
    for t in range(ntile):
        qt = q[t * TS:(t + 1) * TS, :]
        st = lax.dot_general(qt, k, (((1,), (1,)), ((), ())),
                             precision=PREC, preferred_element_type=F32)
        ct = c_ref[t * TS:(t + 1) * TS, :]
        ssum = jnp.sum(st * ct.astype(F32), axis=1, keepdims=True)
        smax = jnp.max(jnp.where(ct > 0, st, -jnp.inf), axis=1, keepdims=True)
        cols.append(smax - ssum / Lx)
    M = jnp.concatenate(cols, axis=1)
    idxmat = (lax.broadcasted_iota(jnp.int32, (TS, ntile), 0)
              + TS * lax.broadcasted_iota(jnp.int32, (TS, ntile), 1))

    U_PAD = 24
    iota_row = lax.broadcasted_iota(jnp.int32, (1, Lx), 1)
    oh_rows = []
    for i in range(u):
        m = jnp.max(M)
        first = jnp.min(jnp.where(M == m, idxmat, Lx))
        oh_rows.append((iota_row == first).astype(F32))
        M = jnp.where(idxmat == first, -jnp.inf, M)
    for i in range(u, U_PAD):
        oh_rows.append(jnp.zeros((1, Lx), F32))
    onehot = jnp.concatenate(oh_rows, axis=0)

    q_red = jnp.dot(onehot, q, precision=PREC, preferred_element_type=F32)
    scores = lax.dot_general(q_red, k, (((1,), (1,)), ((), ())),
                             precision=PREC, preferred_element_type=F32)
    scores = scores / jnp.sqrt(jnp.float32(DK))
    smax = jnp.max(scores, axis=1, keepdims=True)
    e = jnp.exp(scores - smax)
    attn = e / jnp.sum(e, axis=1, keepdims=True)
    upd = jnp.dot(attn, v, precision=PREC, preferred_element_type=F32)

    mean_v = jnp.mean(v, axis=0, keepdims=True)
    selcol = lax.dot_general(onehot, jnp.ones((U_PAD, 1), F32),
                             (((0,), (0,)), ((), ())),
                             precision=PREC, preferred_element_type=F32)
    scat = lax.dot_general(onehot, upd, (((0,), (0,)), ((), ())),
                           precision=PREC, preferred_element_type=F32)
    return mean_v * (1.0 - selcol) + scat


def _attn_body(q_ref, k_ref, v_ref, c_ref, o_ref, *, Lx, u):
    parts = []
    for s in range(2):
        sl = slice(s * DK, (s + 1) * DK)
        parts.append(_attn_one_head(q_ref[:, sl], k_ref[:, sl], v_ref[:, sl],
                                    c_ref, Lx=Lx, u=u))
    o_ref[...] = jnp.concatenate(parts, axis=1)


def _attention(qkv, layer_idx, Lx):
    u, cmat = _sample_constants(layer_idx, Lx)
    body = functools.partial(_attn_body, Lx=Lx, u=u)
    HP = N_HEADS // 2
    return pl.pallas_call(
        body,
        grid=(B, HP),
        in_specs=[
            pl.BlockSpec((Lx, 2 * DK), lambda b, h: (b, h)),
            pl.BlockSpec((Lx, 2 * DK), lambda b, h: (b, HP + h)),
            pl.BlockSpec((Lx, 2 * DK), lambda b, h: (b, 2 * HP + h)),
            pl.BlockSpec((Lx, Lx), lambda b, h: (0, 0)),
        ],
        out_specs=pl.BlockSpec((Lx, 2 * DK), lambda b, h: (b, h)),
        out_shape=jax.ShapeDtypeStruct((B * Lx, D_MODEL), F32),
    )(qkv, qkv, qkv, cmat)


def _ln(x, g, b):
    m = jnp.mean(x, axis=-1, keepdims=True)
    v = jnp.mean((x - m) ** 2, axis=-1, keepdims=True)
    return (x - m) / jnp.sqrt(v + 1e-5) * g + b


def _dense_body(xin_ref, ctx_ref, wo_ref, bo_ref, g1_ref, b1n_ref,
                w1_ref, b1_ref, w2_ref, b2_ref, g2_ref, b2n_ref, o_ref):
    a = jnp.dot(ctx_ref[...], wo_ref[...], precision=PREC,
                preferred_element_type=F32) + bo_ref[...]
    x = xin_ref[...] + a
    xn = _ln(x, g1_ref[...], b1n_ref[...])
    y = jnp.dot(xn, w1_ref[...], precision=PREC, preferred_element_type=F32) + b1_ref[...]
    y = y * 0.5 * (1.0 + lax.erf(y * (1.0 / np.sqrt(2.0).astype(np.float32))))
    z = jnp.dot(y, w2_ref[...], precision=PREC, preferred_element_type=F32) + b2_ref[...]
    o_ref[...] = _ln(xn + z, g2_ref[...], b2n_ref[...])


def _dense(x_flat, ctx_flat, lp, rows):
    TM = 256
    nt = rows // TM
    wo = lp["Wo"].T
    w1 = lp["conv1_w"][:, :, 0].T
    w2 = lp["conv2_w"][:, :, 0].T
    vec = lambda a: a[None, :]
    return pl.pallas_call(
        _dense_body,
        grid=(nt,),
        in_specs=[
            pl.BlockSpec((TM, D_MODEL), lambda i: (i, 0)),
            pl.BlockSpec((TM, D_MODEL), lambda i: (i, 0)),
            pl.BlockSpec((D_MODEL, D_MODEL), lambda i: (0, 0)),
            pl.BlockSpec((1, D_MODEL), lambda i: (0, 0)),
            pl.BlockSpec((1, D_MODEL), lambda i: (0, 0)),
            pl.BlockSpec((1, D_MODEL), lambda i: (0, 0)),
            pl.BlockSpec((D_MODEL, D_FF), lambda i: (0, 0)),
            pl.BlockSpec((1, D_FF), lambda i: (0, 0)),
            pl.BlockSpec((D_FF, D_MODEL), lambda i: (0, 0)),
            pl.BlockSpec((1, D_MODEL), lambda i: (0, 0)),
            pl.BlockSpec((1, D_MODEL), lambda i: (0, 0)),
            pl.BlockSpec((1, D_MODEL), lambda i: (0, 0)),
        ],
        out_specs=pl.BlockSpec((TM, D_MODEL), lambda i: (i, 0)),
        out_shape=jax.ShapeDtypeStruct((rows, D_MODEL), F32),
    )(x_flat, ctx_flat, wo, vec(lp["bo"]), vec(lp["n1_g"]), vec(lp["n1_b"]),
      w1, vec(lp["conv1_b"]), w2, vec(lp["conv2_b"]),
      vec(lp["n2_g"]), vec(lp["n2_b"]))


def _distill_body(xe_ref, xo_ref, xom1_ref, xem1_ref, xep1_ref,
                  w0_ref, w1_ref, w2_ref, a_ref, bt_ref, o_ref, *, TM):
    w0 = w0_ref[...]
    w1 = w1_ref[...]
    w2 = w2_ref[...]
    a = a_ref[...]
    bt = bt_ref[...]

    def conv(xa, xb, xc):
        h = (jnp.dot(xa, w0, precision=PREC, preferred_element_type=F32)
             + jnp.dot(xb, w1, precision=PREC, preferred_element_type=F32)
             + jnp.dot(xc, w2, precision=PREC, preferred_element_type=F32))
        h = h * a + bt
        return jnp.where(h > 0, h, jnp.exp(jnp.minimum(h, 0.0)) - 1.0)

    he = conv(xom1_ref[...], xe_ref[...], xo_ref[...])
    ho = conv(xe_ref[...], xo_ref[...], xep1_ref[...])
    hm = conv(xem1_ref[...], xom1_ref[...], xe_ref[...])
    grow = pl.program_id(1) * TM + lax.broadcasted_iota(jnp.int32, hm.shape, 0)
    hm = jnp.where(grow == 0, -jnp.inf, hm)
    o_ref[...] = jnp.maximum(jnp.maximum(hm, he), ho)


def _distill(x_flat, cp, Lx):
    L2 = Lx // 2
    x4 = x_flat.reshape(B, L2, 2, D_MODEL)
    xe = x4[:, :, 0, :]
    xo = x4[:, :, 1, :]
    xo_m1 = jnp.roll(xo, 1, axis=1)
    xe_m1 = jnp.roll(xe, 1, axis=1)
    xe_p1 = jnp.roll(xe, -1, axis=1)
    flat = lambda t: t.reshape(B * L2, D_MODEL)
    w0 = cp["dc_w"][:, :, 0].T
    w1 = cp["dc_w"][:, :, 1].T
    w2 = cp["dc_w"][:, :, 2].T
    a = (cp["bn_g"] / jnp.sqrt(1.0 + 1e-5))[None, :]
    bt = cp["dc_b"][None, :] * a + cp["bn_b"][None, :]
    TM = 128
    nt = L2 // TM
    row_spec = pl.BlockSpec((TM, D_MODEL), lambda b, t: (b * nt + t, 0))
    w_spec = pl.BlockSpec((D_MODEL, D_MODEL), lambda b, t: (0, 0))
    v_spec = pl.BlockSpec((1, D_MODEL), lambda b, t: (0, 0))
    return pl.pallas_call(
        functools.partial(_distill_body, TM=TM),
        grid=(B, nt),
        in_specs=[row_spec] * 5 + [w_spec] * 3 + [v_spec] * 2,
        out_specs=row_spec,
        out_shape=jax.ShapeDtypeStruct((B * L2, D_MODEL), F32),
    )(flat(xe), flat(xo), flat(xo_m1), flat(xe_m1), flat(xe_p1),
      w0, w1, w2, a, bt)


def _final_body(x_ref, g_ref, b_ref, o_ref):
    xn = _ln(x_ref[...], g_ref[...], b_ref[...])
    o_ref[...] = jnp.mean(xn, axis=0, keepdims=True)[None]


def _final(x_flat, g, b, Lx):
    out = pl.pallas_call(
        _final_body,
        grid=(B,),
        in_specs=[
            pl.BlockSpec((Lx, D_MODEL), lambda i: (i, 0)),
            pl.BlockSpec((1, D_MODEL), lambda i: (0, 0)),
            pl.BlockSpec((1, D_MODEL), lambda i: (0, 0)),
        ],
        out_specs=pl.BlockSpec((1, 1, D_MODEL), lambda i: (i, 0, 0)),
        out_shape=jax.ShapeDtypeStruct((B, 1, D_MODEL), F32),
    )(x_flat, g[None, :], b[None, :])
    return out.reshape(B, D_MODEL)


def kernel(x_enc, params):
    x = _embed(x_enc.reshape(B * L0, C_IN), params["tok_conv_w"])
    Lx = L0
    for l in range(E_LAYERS):
        rows = B * Lx
        qkv = _qkv(x, params["layers"][l], rows)
        ctx = _attention(qkv, l, Lx)
        x = _dense(x, ctx, params["layers"][l], rows)
        if l < E_LAYERS - 1:
            x = _distill(x, params["convs"][l], Lx)
            Lx //= 2
    return _final(x, params["norm_g"], params["norm_b"], Lx)


# speedup vs baseline: 1.2040x; 1.2040x over previous
import functools
import math

import numpy as np
import jax
import jax.numpy as jnp
from jax import lax
from jax.experimental import pallas as pl
from jax.experimental.pallas import tpu as pltpu

B, L0, C_IN = 2, 2048, 7
D_MODEL, N_HEADS, E_LAYERS = 1024, 16, 3
D_FF, FACTOR = 512, 3
DK = D_MODEL // N_HEADS

PREC = lax.Precision.DEFAULT
F32 = jnp.float32

def _sample_constants_eager():
    consts = []
    base = jax.random.key(42)
    for l in range(E_LAYERS):
        Ll = L0 >> l
        u = min(int(FACTOR * np.ceil(np.log(Ll))), Ll)
        idx = np.asarray(jax.random.randint(jax.random.fold_in(base, l),
                                            (Ll, u), 0, Ll))
        cnt = np.zeros((Ll, Ll), np.float32)
        np.add.at(cnt, (np.arange(Ll)[:, None], idx), 1.0)
        consts.append((u, cnt.astype(jnp.bfloat16)))
    return consts


with jax.default_device(jax.local_devices(backend="cpu")[0]):
    _SAMPLE_CONSTS = _sample_constants_eager()


def _sample_constants(layer_idx, Ll):
    u, cnt = _SAMPLE_CONSTS[layer_idx]
    return u, jnp.asarray(cnt)


def _embed_body(x_ref, w0_ref, w1_ref, w2_ref, o_ref):
    x = x_ref[...]
    xm1 = jnp.concatenate([x[-1:, :], x[:-1, :]], axis=0)
    xp1 = jnp.concatenate([x[1:, :], x[:1, :]], axis=0)
    o_ref[...] = (
        jnp.dot(xm1, w0_ref[...], precision=PREC, preferred_element_type=F32)
        + jnp.dot(x, w1_ref[...], precision=PREC, preferred_element_type=F32)
        + jnp.dot(xp1, w2_ref[...], precision=PREC, preferred_element_type=F32)
    )


def _embed(x_flat, w):
    w0 = w[:, :, 0].T
    w1 = w[:, :, 1].T
    w2 = w[:, :, 2].T
    return pl.pallas_call(
        _embed_body,
        grid=(B,),
        in_specs=[
            pl.BlockSpec((L0, C_IN), lambda b: (b, 0)),
            pl.BlockSpec((C_IN, D_MODEL), lambda b: (0, 0)),
            pl.BlockSpec((C_IN, D_MODEL), lambda b: (0, 0)),
            pl.BlockSpec((C_IN, D_MODEL), lambda b: (0, 0)),
        ],
        out_specs=pl.BlockSpec((L0, D_MODEL), lambda b: (b, 0)),
        out_shape=jax.ShapeDtypeStruct((B * L0, D_MODEL), F32),
    )(x_flat, w0, w1, w2)


def _qkv_body(x_ref, w_ref, b_ref, o_ref):
    o_ref[...] = (
        jnp.dot(x_ref[...], w_ref[...], precision=PREC, preferred_element_type=F32)
        + b_ref[...]
    )


def _qkv(x_flat, lp, rows):
    w = jnp.concatenate([lp["Wq"], lp["Wk"], lp["Wv"]], axis=0).T
    b = jnp.concatenate([lp["bq"], lp["bk"], lp["bv"]])[None, :]
    TM = 256
    nt = rows // TM
    return pl.pallas_call(
        _qkv_body,
        grid=(nt,),
        in_specs=[
            pl.BlockSpec((TM, D_MODEL), lambda i: (i, 0)),
            pl.BlockSpec((D_MODEL, 3 * D_MODEL), lambda i: (0, 0)),
            pl.BlockSpec((1, 3 * D_MODEL), lambda i: (0, 0)),
        ],
        out_specs=pl.BlockSpec((TM, 3 * D_MODEL), lambda i: (i, 0)),
        out_shape=jax.ShapeDtypeStruct((rows, 3 * D_MODEL), F32),
    )(x_flat, w, b)


def _attn_one_head(q, k, v, c_ref, *, Lx, u):
    TS = 256
    ntile = Lx // TS

    cols = []
    for t in range(ntile):
        qt = q[t * TS:(t + 1) * TS, :]
        st = lax.dot_general(qt, k, (((1,), (1,)), ((), ())),
                             precision=PREC, preferred_element_type=F32)
        ct = c_ref[t * TS:(t + 1) * TS, :]
        ssum = jnp.sum(st * ct.astype(F32), axis=1, keepdims=True)
        smax = jnp.max(jnp.where(ct > 0, st, -jnp.inf), axis=1, keepdims=True)
        cols.append(smax - ssum / Lx)
    M = jnp.concatenate(cols, axis=1)
    idxmat = (lax.broadcasted_iota(jnp.int32, (TS, ntile), 0)
              + TS * lax.broadcasted_iota(jnp.int32, (TS, ntile), 1))

    U_PAD = 24
    iota_row = lax.broadcasted_iota(jnp.int32, (1, Lx), 1)
    oh_rows = []
    for i in range(u):
        m = jnp.max(M)
        first = jnp.min(jnp.where(M == m, idxmat, Lx))
        oh_rows.append((iota_row == first).astype(F32))
        M = jnp.where(idxmat == first, -jnp.inf, M)
    for i in range(u, U_PAD):
        oh_rows.append(jnp.zeros((1, Lx), F32))
    onehot = jnp.concatenate(oh_rows, axis=0)

    q_red = jnp.dot(onehot, q, precision=PREC, preferred_element_type=F32)
    scores = lax.dot_general(q_red, k, (((1,), (1,)), ((), ())),
                             precision=PREC, preferred_element_type=F32)
    scores = scores / jnp.sqrt(jnp.float32(DK))
    smax = jnp.max(scores, axis=1, keepdims=True)
    e = jnp.exp(scores - smax)
    attn = e / jnp.sum(e, axis=1, keepdims=True)
    upd = jnp.dot(attn, v, precision=PREC, preferred_element_type=F32)

    mean_v = jnp.mean(v, axis=0, keepdims=True)
    selcol = lax.dot_general(onehot, jnp.ones((U_PAD, 1), F32),
                             (((0,), (0,)), ((), ())),
                             precision=PREC, preferred_element_type=F32)
    scat = lax.dot_general(onehot, upd, (((0,), (0,)), ((), ())),
                           precision=PREC, preferred_element_type=F32)
    return mean_v * (1.0 - selcol) + scat


def _attn_body(q_ref, k_ref, v_ref, c_ref, o_ref, *, Lx, u):
    parts = []
    for s in range(2):
        sl = slice(s * DK, (s + 1) * DK)
        parts.append(_attn_one_head(q_ref[:, sl], k_ref[:, sl], v_ref[:, sl],
                                    c_ref, Lx=Lx, u=u))
    o_ref[...] = jnp.concatenate(parts, axis=1)


def _attention(qkv, layer_idx, Lx):
    u, cmat = _sample_constants(layer_idx, Lx)
    body = functools.partial(_attn_body, Lx=Lx, u=u)
    HP = N_HEADS // 2
    return pl.pallas_call(
        body,
        grid=(B, HP),
        in_specs=[
            pl.BlockSpec((Lx, 2 * DK), lambda b, h: (b, h)),
            pl.BlockSpec((Lx, 2 * DK), lambda b, h: (b, HP + h)),
            pl.BlockSpec((Lx, 2 * DK), lambda b, h: (b, 2 * HP + h)),
            pl.BlockSpec((Lx, Lx), lambda b, h: (0, 0)),
        ],
        out_specs=pl.BlockSpec((Lx, 2 * DK), lambda b, h: (b, h)),
        out_shape=jax.ShapeDtypeStruct((B * Lx, D_MODEL), F32),
    )(qkv, qkv, qkv, cmat)


def _ln(x, g, b):
    m = jnp.mean(x, axis=-1, keepdims=True)
    v = jnp.mean((x - m) ** 2, axis=-1, keepdims=True)
    return (x - m) / jnp.sqrt(v + 1e-5) * g + b


def _dense_body(xin_ref, ctx_ref, wo_ref, bo_ref, g1_ref, b1n_ref,
                w1_ref, b1_ref, w2_ref, b2_ref, g2_ref, b2n_ref, o_ref):
    a = jnp.dot(ctx_ref[...], wo_ref[...], precision=PREC,
                preferred_element_type=F32) + bo_ref[...]
    x = xin_ref[...] + a
    xn = _ln(x, g1_ref[...], b1n_ref[...])
    y = jnp.dot(xn, w1_ref[...], precision=PREC, preferred_element_type=F32) + b1_ref[...]
    y = y * 0.5 * (1.0 + lax.erf(y * (1.0 / np.sqrt(2.0).astype(np.float32))))
    z = jnp.dot(y, w2_ref[...], precision=PREC, preferred_element_type=F32) + b2_ref[...]
    o_ref[...] = _ln(xn + z, g2_ref[...], b2n_ref[...])


def _dense(x_flat, ctx_flat, lp, rows):
    TM = 256
    nt = rows // TM
    wo = lp["Wo"].T
    w1 = lp["conv1_w"][:, :, 0].T
    w2 = lp["conv2_w"][:, :, 0].T
    vec = lambda a: a[None, :]
    return pl.pallas_call(
        _dense_body,
        grid=(nt,),
        in_specs=[
            pl.BlockSpec((TM, D_MODEL), lambda i: (i, 0)),
            pl.BlockSpec((TM, D_MODEL), lambda i: (i, 0)),
            pl.BlockSpec((D_MODEL, D_MODEL), lambda i: (0, 0)),
            pl.BlockSpec((1, D_MODEL), lambda i: (0, 0)),
            pl.BlockSpec((1, D_MODEL), lambda i: (0, 0)),
            pl.BlockSpec((1, D_MODEL), lambda i: (0, 0)),
            pl.BlockSpec((D_MODEL, D_FF), lambda i: (0, 0)),
            pl.BlockSpec((1, D_FF), lambda i: (0, 0)),
            pl.BlockSpec((D_FF, D_MODEL), lambda i: (0, 0)),
            pl.BlockSpec((1, D_MODEL), lambda i: (0, 0)),
            pl.BlockSpec((1, D_MODEL), lambda i: (0, 0)),
            pl.BlockSpec((1, D_MODEL), lambda i: (0, 0)),
        ],
        out_specs=pl.BlockSpec((TM, D_MODEL), lambda i: (i, 0)),
        out_shape=jax.ShapeDtypeStruct((rows, D_MODEL), F32),
    )(x_flat, ctx_flat, wo, vec(lp["bo"]), vec(lp["n1_g"]), vec(lp["n1_b"]),
      w1, vec(lp["conv1_b"]), w2, vec(lp["conv2_b"]),
      vec(lp["n2_g"]), vec(lp["n2_b"]))


def _distill_body(xe_ref, xo_ref, xom1_ref, xem1_ref, xep1_ref,
                  w0_ref, w1_ref, w2_ref, a_ref, bt_ref, o_ref, *, TM):
    w0 = w0_ref[...]
    w1 = w1_ref[...]
    w2 = w2_ref[...]
    a = a_ref[...]
    bt = bt_ref[...]

    def conv(xa, xb, xc):
        h = (jnp.dot(xa, w0, precision=PREC, preferred_element_type=F32)
             + jnp.dot(xb, w1, precision=PREC, preferred_element_type=F32)
             + jnp.dot(xc, w2, precision=PREC, preferred_element_type=F32))
        h = h * a + bt
        return jnp.where(h > 0, h, jnp.exp(jnp.minimum(h, 0.0)) - 1.0)

    he = conv(xom1_ref[...], xe_ref[...], xo_ref[...])
    ho = conv(xe_ref[...], xo_ref[...], xep1_ref[...])
    hm = conv(xem1_ref[...], xom1_ref[...], xe_ref[...])
    grow = pl.program_id(1) * TM + lax.broadcasted_iota(jnp.int32, hm.shape, 0)
    hm = jnp.where(grow == 0, -jnp.inf, hm)
    o_ref[...] = jnp.maximum(jnp.maximum(hm, he), ho)


def _distill(x_flat, cp, Lx):
    L2 = Lx // 2
    x4 = x_flat.reshape(B, L2, 2, D_MODEL)
    xe = x4[:, :, 0, :]
    xo = x4[:, :, 1, :]
    xo_m1 = jnp.roll(xo, 1, axis=1)
    xe_m1 = jnp.roll(xe, 1, axis=1)
    xe_p1 = jnp.roll(xe, -1, axis=1)
    flat = lambda t: t.reshape(B * L2, D_MODEL)
    w0 = cp["dc_w"][:, :, 0].T
    w1 = cp["dc_w"][:, :, 1].T
    w2 = cp["dc_w"][:, :, 2].T
    a = (cp["bn_g"] / jnp.sqrt(1.0 + 1e-5))[None, :]
    bt = cp["dc_b"][None, :] * a + cp["bn_b"][None, :]
    TM = 128
    nt = L2 // TM
    row_spec = pl.BlockSpec((TM, D_MODEL), lambda b, t: (b * nt + t, 0))
    w_spec = pl.BlockSpec((D_MODEL, D_MODEL), lambda b, t: (0, 0))
    v_spec = pl.BlockSpec((1, D_MODEL), lambda b, t: (0, 0))
    return pl.pallas_call(
        functools.partial(_distill_body, TM=TM),
        grid=(B, nt),
        in_specs=[row_spec] * 5 + [w_spec] * 3 + [v_spec] * 2,
        out_specs=row_spec,
        out_shape=jax.ShapeDtypeStruct((B * L2, D_MODEL), F32),
    )(flat(xe), flat(xo), flat(xo_m1), flat(xe_m1), flat(xe_p1),
      w0, w1, w2, a, bt)


def _final_body(x_ref, g_ref, b_ref, o_ref):
    xn = _ln(x_ref[...], g_ref[...], b_ref[...])
    o_ref[...] = jnp.mean(xn, axis=0, keepdims=True)[None]


def _final(x_flat, g, b, Lx):
    out = pl.pallas_call(
        _final_body,
        grid=(B,),
        in_specs=[
            pl.BlockSpec((Lx, D_MODEL), lambda i: (i, 0)),
            pl.BlockSpec((1, D_MODEL), lambda i: (0, 0)),
            pl.BlockSpec((1, D_MODEL), lambda i: (0, 0)),
        ],
        out_specs=pl.BlockSpec((1, 1, D_MODEL), lambda i: (i, 0, 0)),
        out_shape=jax.ShapeDtypeStruct((B, 1, D_MODEL), F32),
    )(x_flat, g[None, :], b[None, :])
    return out.reshape(B, D_MODEL)


def kernel(x_enc, params):
    x = _embed(x_enc.reshape(B * L0, C_IN), params["tok_conv_w"])
    Lx = L0
    for l in range(E_LAYERS):
        rows = B * Lx
        qkv = _qkv(x, params["layers"][l], rows)
        ctx = _attention(qkv, l, Lx)
        x = _dense(x, ctx, params["layers"][l], rows)
        if l < E_LAYERS - 1:
            x = _distill(x, params["convs"][l], Lx)
            Lx //= 2
    return _final(x, params["norm_g"], params["norm_b"], Lx)


# speedup vs baseline: 1.9474x; 1.6175x over previous
import functools
import math

import numpy as np
import jax
import jax.numpy as jnp
from jax import lax
from jax.experimental import pallas as pl
from jax.experimental.pallas import tpu as pltpu

B, L0, C_IN = 2, 2048, 7
D_MODEL, N_HEADS, E_LAYERS = 1024, 16, 3
D_FF, FACTOR = 512, 3
DK = D_MODEL // N_HEADS

PREC = lax.Precision.DEFAULT
F32 = jnp.float32

def _sample_constants_eager():
    consts = []
    base = jax.random.key(42)
    for l in range(E_LAYERS):
        Ll = L0 >> l
        u = min(int(FACTOR * np.ceil(np.log(Ll))), Ll)
        idx = np.asarray(jax.random.randint(jax.random.fold_in(base, l),
                                            (Ll, u), 0, Ll))
        cnt = np.zeros((Ll, Ll), np.float32)
        np.add.at(cnt, (np.arange(Ll)[:, None], idx), 1.0)
        consts.append((u, cnt.astype(jnp.bfloat16)))
    return consts


with jax.default_device(jax.local_devices(backend="cpu")[0]):
    _SAMPLE_CONSTS = _sample_constants_eager()


def _sample_constants(layer_idx, Ll):
    u, cnt = _SAMPLE_CONSTS[layer_idx]
    return u, jnp.asarray(cnt)


def _embed_body(x_ref, w0_ref, w1_ref, w2_ref, o_ref):
    x = x_ref[...]
    xm1 = jnp.concatenate([x[-1:, :], x[:-1, :]], axis=0)
    xp1 = jnp.concatenate([x[1:, :], x[:1, :]], axis=0)
    o_ref[...] = (
        jnp.dot(xm1, w0_ref[...], precision=PREC, preferred_element_type=F32)
        + jnp.dot(x, w1_ref[...], precision=PREC, preferred_element_type=F32)
        + jnp.dot(xp1, w2_ref[...], precision=PREC, preferred_element_type=F32)
    )


def _embed(x_flat, w):
    w0 = w[:, :, 0].T
    w1 = w[:, :, 1].T
    w2 = w[:, :, 2].T
    return pl.pallas_call(
        _embed_body,
        grid=(B,),
        in_specs=[
            pl.BlockSpec((L0, C_IN), lambda b: (b, 0)),
            pl.BlockSpec((C_IN, D_MODEL), lambda b: (0, 0)),
            pl.BlockSpec((C_IN, D_MODEL), lambda b: (0, 0)),
            pl.BlockSpec((C_IN, D_MODEL), lambda b: (0, 0)),
        ],
        out_specs=pl.BlockSpec((L0, D_MODEL), lambda b: (b, 0)),
        out_shape=jax.ShapeDtypeStruct((B * L0, D_MODEL), F32),
    )(x_flat, w0, w1, w2)


def _qkv_body(x_ref, w_ref, b_ref, o_ref):
    o_ref[...] = (
        jnp.dot(x_ref[...], w_ref[...], precision=PREC, preferred_element_type=F32)
        + b_ref[...]
    )


def _qkv(x_flat, lp, rows):
    w = jnp.concatenate([lp["Wq"], lp["Wk"], lp["Wv"]], axis=0).T
    b = jnp.concatenate([lp["bq"], lp["bk"], lp["bv"]])[None, :]
    TM = 256
    nt = rows // TM
    return pl.pallas_call(
        _qkv_body,
        grid=(nt,),
        in_specs=[
            pl.BlockSpec((TM, D_MODEL), lambda i: (i, 0)),
            pl.BlockSpec((D_MODEL, 3 * D_MODEL), lambda i: (0, 0)),
            pl.BlockSpec((1, 3 * D_MODEL), lambda i: (0, 0)),
        ],
        out_specs=pl.BlockSpec((TM, 3 * D_MODEL), lambda i: (i, 0)),
        out_shape=jax.ShapeDtypeStruct((rows, 3 * D_MODEL), F32),
    )(x_flat, w, b)


def _attn_one_head(q, k, v, c_ref, *, Lx, u):
    TS = 256
    ntile = Lx // TS

    cols = []
    for t in range(ntile):
        qt = q[t * TS:(t + 1) * TS, :]
        st = lax.dot_general(qt, k, (((1,), (1,)), ((), ())),
                             precision=PREC, preferred_element_type=F32)
        ct = c_ref[t * TS:(t + 1) * TS, :]
        ssum = jnp.sum(st * ct.astype(F32), axis=1, keepdims=True)
        smax = jnp.max(jnp.where(ct > 0, st, -jnp.inf), axis=1, keepdims=True)
        cols.append(smax - ssum / Lx)
    M = jnp.concatenate(cols, axis=1)
    m_row = jnp.reshape(jnp.transpose(M), (1, Lx))
    iota_lane = lax.broadcasted_iota(jnp.int32, (1, Lx), 1)
    BF = jnp.bfloat16
    ones_l = jnp.ones((Lx, 1), BF)
    ranks = []
    for t in range(ntile):
        m_col = cols[t]
        idx_col = (t * TS
                   + lax.broadcasted_iota(jnp.int32, (TS, 1), 0))
        cntf = ((m_row > m_col).astype(BF)
                + ((m_row == m_col) & (iota_lane < idx_col)).astype(BF))
        ranks.append(lax.dot_general(cntf, ones_l, (((1,), (0,)), ((), ())),
                                     preferred_element_type=F32))
    rank_col = jnp.concatenate(ranks, axis=0)

    U_PAD = 24
    slot = lax.broadcasted_iota(jnp.int32, (1, U_PAD), 1)
    onehot_t = ((rank_col == slot.astype(F32)) & (slot < u)).astype(F32)

    q_red = lax.dot_general(onehot_t, q, (((0,), (0,)), ((), ())),
                            precision=PREC, preferred_element_type=F32)
    scores = lax.dot_general(q_red, k, (((1,), (1,)), ((), ())),
                             precision=PREC, preferred_element_type=F32)
    scores = scores / jnp.sqrt(jnp.float32(DK))
    smax = jnp.max(scores, axis=1, keepdims=True)
    e = jnp.exp(scores - smax)
    attn = e / jnp.sum(e, axis=1, keepdims=True)
    upd = jnp.dot(attn, v, precision=PREC, preferred_element_type=F32)

    mean_v = jnp.mean(v, axis=0, keepdims=True)
    selcol = jnp.sum(onehot_t, axis=1, keepdims=True)
    scat = jnp.dot(onehot_t, upd, precision=PREC,
                   preferred_element_type=F32)
    return mean_v * (1.0 - selcol) + scat


def _attn_body(q_ref, k_ref, v_ref, c_ref, o_ref, *, Lx, u):
    parts = []
    for s in range(2):
        sl = slice(s * DK, (s + 1) * DK)
        parts.append(_attn_one_head(q_ref[:, sl], k_ref[:, sl], v_ref[:, sl],
                                    c_ref, Lx=Lx, u=u))
    o_ref[...] = jnp.concatenate(parts, axis=1)


def _attention(qkv, layer_idx, Lx):
    u, cmat = _sample_constants(layer_idx, Lx)
    body = functools.partial(_attn_body, Lx=Lx, u=u)
    HP = N_HEADS // 2
    return pl.pallas_call(
        body,
        grid=(B, HP),
        in_specs=[
            pl.BlockSpec((Lx, 2 * DK), lambda b, h: (b, h)),
            pl.BlockSpec((Lx, 2 * DK), lambda b, h: (b, HP + h)),
            pl.BlockSpec((Lx, 2 * DK), lambda b, h: (b, 2 * HP + h)),
            pl.BlockSpec((Lx, Lx), lambda b, h: (0, 0)),
        ],
        out_specs=pl.BlockSpec((Lx, 2 * DK), lambda b, h: (b, h)),
        out_shape=jax.ShapeDtypeStruct((B * Lx, D_MODEL), F32),
    )(qkv, qkv, qkv, cmat)


def _ln(x, g, b):
    m = jnp.mean(x, axis=-1, keepdims=True)
    v = jnp.mean((x - m) ** 2, axis=-1, keepdims=True)
    return (x - m) / jnp.sqrt(v + 1e-5) * g + b


def _dense_body(xin_ref, ctx_ref, wo_ref, bo_ref, g1_ref, b1n_ref,
                w1_ref, b1_ref, w2_ref, b2_ref, g2_ref, b2n_ref, o_ref):
    a = jnp.dot(ctx_ref[...], wo_ref[...], precision=PREC,
                preferred_element_type=F32) + bo_ref[...]
    x = xin_ref[...] + a
    xn = _ln(x, g1_ref[...], b1n_ref[...])
    y = jnp.dot(xn, w1_ref[...], precision=PREC, preferred_element_type=F32) + b1_ref[...]
    y = y * 0.5 * (1.0 + lax.erf(y * (1.0 / np.sqrt(2.0).astype(np.float32))))
    z = jnp.dot(y, w2_ref[...], precision=PREC, preferred_element_type=F32) + b2_ref[...]
    o_ref[...] = _ln(xn + z, g2_ref[...], b2n_ref[...])


def _dense(x_flat, ctx_flat, lp, rows):
    TM = 256
    nt = rows // TM
    wo = lp["Wo"].T
    w1 = lp["conv1_w"][:, :, 0].T
    w2 = lp["conv2_w"][:, :, 0].T
    vec = lambda a: a[None, :]
    return pl.pallas_call(
        _dense_body,
        grid=(nt,),
        in_specs=[
            pl.BlockSpec((TM, D_MODEL), lambda i: (i, 0)),
            pl.BlockSpec((TM, D_MODEL), lambda i: (i, 0)),
            pl.BlockSpec((D_MODEL, D_MODEL), lambda i: (0, 0)),
            pl.BlockSpec((1, D_MODEL), lambda i: (0, 0)),
            pl.BlockSpec((1, D_MODEL), lambda i: (0, 0)),
            pl.BlockSpec((1, D_MODEL), lambda i: (0, 0)),
            pl.BlockSpec((D_MODEL, D_FF), lambda i: (0, 0)),
            pl.BlockSpec((1, D_FF), lambda i: (0, 0)),
            pl.BlockSpec((D_FF, D_MODEL), lambda i: (0, 0)),
            pl.BlockSpec((1, D_MODEL), lambda i: (0, 0)),
            pl.BlockSpec((1, D_MODEL), lambda i: (0, 0)),
            pl.BlockSpec((1, D_MODEL), lambda i: (0, 0)),
        ],
        out_specs=pl.BlockSpec((TM, D_MODEL), lambda i: (i, 0)),
        out_shape=jax.ShapeDtypeStruct((rows, D_MODEL), F32),
    )(x_flat, ctx_flat, wo, vec(lp["bo"]), vec(lp["n1_g"]), vec(lp["n1_b"]),
      w1, vec(lp["conv1_b"]), w2, vec(lp["conv2_b"]),
      vec(lp["n2_g"]), vec(lp["n2_b"]))


def _distill_body(xe_ref, xo_ref, xom1_ref, xem1_ref, xep1_ref,
                  w0_ref, w1_ref, w2_ref, a_ref, bt_ref, o_ref, *, TM):
    w0 = w0_ref[...]
    w1 = w1_ref[...]
    w2 = w2_ref[...]
    a = a_ref[...]
    bt = bt_ref[...]

    def conv(xa, xb, xc):
        h = (jnp.dot(xa, w0, precision=PREC, preferred_element_type=F32)
             + jnp.dot(xb, w1, precision=PREC, preferred_element_type=F32)
             + jnp.dot(xc, w2, precision=PREC, preferred_element_type=F32))
        h = h * a + bt
        return jnp.where(h > 0, h, jnp.exp(jnp.minimum(h, 0.0)) - 1.0)

    he = conv(xom1_ref[...], xe_ref[...], xo_ref[...])
    ho = conv(xe_ref[...], xo_ref[...], xep1_ref[...])
    hm = conv(xem1_ref[...], xom1_ref[...], xe_ref[...])
    grow = pl.program_id(1) * TM + lax.broadcasted_iota(jnp.int32, hm.shape, 0)
    hm = jnp.where(grow == 0, -jnp.inf, hm)
    o_ref[...] = jnp.maximum(jnp.maximum(hm, he), ho)


def _distill(x_flat, cp, Lx):
    L2 = Lx // 2
    x4 = x_flat.reshape(B, L2, 2, D_MODEL)
    xe = x4[:, :, 0, :]
    xo = x4[:, :, 1, :]
    xo_m1 = jnp.roll(xo, 1, axis=1)
    xe_m1 = jnp.roll(xe, 1, axis=1)
    xe_p1 = jnp.roll(xe, -1, axis=1)
    flat = lambda t: t.reshape(B * L2, D_MODEL)
    w0 = cp["dc_w"][:, :, 0].T
    w1 = cp["dc_w"][:, :, 1].T
    w2 = cp["dc_w"][:, :, 2].T
    a = (cp["bn_g"] / jnp.sqrt(1.0 + 1e-5))[None, :]
    bt = cp["dc_b"][None, :] * a + cp["bn_b"][None, :]
    TM = 128
    nt = L2 // TM
    row_spec = pl.BlockSpec((TM, D_MODEL), lambda b, t: (b * nt + t, 0))
    w_spec = pl.BlockSpec((D_MODEL, D_MODEL), lambda b, t: (0, 0))
    v_spec = pl.BlockSpec((1, D_MODEL), lambda b, t: (0, 0))
    return pl.pallas_call(
        functools.partial(_distill_body, TM=TM),
        grid=(B, nt),
        in_specs=[row_spec] * 5 + [w_spec] * 3 + [v_spec] * 2,
        out_specs=row_spec,
        out_shape=jax.ShapeDtypeStruct((B * L2, D_MODEL), F32),
    )(flat(xe), flat(xo), flat(xo_m1), flat(xe_m1), flat(xe_p1),
      w0, w1, w2, a, bt)


def _final_body(x_ref, g_ref, b_ref, o_ref):
    xn = _ln(x_ref[...], g_ref[...], b_ref[...])
    o_ref[...] = jnp.mean(xn, axis=0, keepdims=True)[None]


def _final(x_flat, g, b, Lx):
    out = pl.pallas_call(
        _final_body,
        grid=(B,),
        in_specs=[
            pl.BlockSpec((Lx, D_MODEL), lambda i: (i, 0)),
            pl.BlockSpec((1, D_MODEL), lambda i: (0, 0)),
            pl.BlockSpec((1, D_MODEL), lambda i: (0, 0)),
        ],
        out_specs=pl.BlockSpec((1, 1, D_MODEL), lambda i: (i, 0, 0)),
        out_shape=jax.ShapeDtypeStruct((B, 1, D_MODEL), F32),
    )(x_flat, g[None, :], b[None, :])
    return out.reshape(B, D_MODEL)


def kernel(x_enc, params):
    x = _embed(x_enc.reshape(B * L0, C_IN), params["tok_conv_w"])
    Lx = L0
    for l in range(E_LAYERS):
        rows = B * Lx
        qkv = _qkv(x, params["layers"][l], rows)
        ctx = _attention(qkv, l, Lx)
        x = _dense(x, ctx, params["layers"][l], rows)
        if l < E_LAYERS - 1:
            x = _distill(x, params["convs"][l], Lx)
            Lx //= 2
    return _final(x, params["norm_g"], params["norm_b"], Lx)


# speedup vs baseline: 2.1015x; 1.0791x over previous
import functools
import math

import numpy as np
import jax
import jax.numpy as jnp
from jax import lax
from jax.experimental import pallas as pl
from jax.experimental.pallas import tpu as pltpu

B, L0, C_IN = 2, 2048, 7
D_MODEL, N_HEADS, E_LAYERS = 1024, 16, 3
D_FF, FACTOR = 512, 3
DK = D_MODEL // N_HEADS

PREC = lax.Precision.DEFAULT
F32 = jnp.float32

def _sample_constants_eager():
    consts = []
    base = jax.random.key(42)
    for l in range(E_LAYERS):
        Ll = L0 >> l
        u = min(int(FACTOR * np.ceil(np.log(Ll))), Ll)
        idx = np.asarray(jax.random.randint(jax.random.fold_in(base, l),
                                            (Ll, u), 0, Ll))
        cnt = np.zeros((Ll, Ll), np.float32)
        np.add.at(cnt, (np.arange(Ll)[:, None], idx), 1.0)
        consts.append((u, cnt.astype(np.int8)))
    return consts


with jax.default_device(jax.local_devices(backend="cpu")[0]):
    _SAMPLE_CONSTS = _sample_constants_eager()


def _sample_constants(layer_idx, Ll):
    u, cnt = _SAMPLE_CONSTS[layer_idx]
    return u, jnp.asarray(cnt)


def _embed_body(x_ref, w0_ref, w1_ref, w2_ref, o_ref):
    x = x_ref[...]
    xm1 = jnp.concatenate([x[-1:, :], x[:-1, :]], axis=0)
    xp1 = jnp.concatenate([x[1:, :], x[:1, :]], axis=0)
    o_ref[...] = (
        jnp.dot(xm1, w0_ref[...], precision=PREC, preferred_element_type=F32)
        + jnp.dot(x, w1_ref[...], precision=PREC, preferred_element_type=F32)
        + jnp.dot(xp1, w2_ref[...], precision=PREC, preferred_element_type=F32)
    )


def _embed(x_flat, w):
    w0 = w[:, :, 0].T
    w1 = w[:, :, 1].T
    w2 = w[:, :, 2].T
    return pl.pallas_call(
        _embed_body,
        grid=(B,),
        in_specs=[
            pl.BlockSpec((L0, C_IN), lambda b: (b, 0)),
            pl.BlockSpec((C_IN, D_MODEL), lambda b: (0, 0)),
            pl.BlockSpec((C_IN, D_MODEL), lambda b: (0, 0)),
            pl.BlockSpec((C_IN, D_MODEL), lambda b: (0, 0)),
        ],
        out_specs=pl.BlockSpec((L0, D_MODEL), lambda b: (b, 0)),
        out_shape=jax.ShapeDtypeStruct((B * L0, D_MODEL), F32),
    )(x_flat, w0, w1, w2)


def _qkv_body(x_ref, w_ref, b_ref, o_ref):
    o_ref[...] = (
        jnp.dot(x_ref[...], w_ref[...], precision=PREC, preferred_element_type=F32)
        + b_ref[...]
    )


def _qkv(x_flat, lp, rows):
    w = jnp.concatenate([lp["Wq"], lp["Wk"], lp["Wv"]], axis=0).T
    b = jnp.concatenate([lp["bq"], lp["bk"], lp["bv"]])[None, :]
    TM = 256
    nt = rows // TM
    return pl.pallas_call(
        _qkv_body,
        grid=(nt,),
        in_specs=[
            pl.BlockSpec((TM, D_MODEL), lambda i: (i, 0)),
            pl.BlockSpec((D_MODEL, 3 * D_MODEL), lambda i: (0, 0)),
            pl.BlockSpec((1, 3 * D_MODEL), lambda i: (0, 0)),
        ],
        out_specs=pl.BlockSpec((TM, 3 * D_MODEL), lambda i: (i, 0)),
        out_shape=jax.ShapeDtypeStruct((rows, 3 * D_MODEL), F32),
    )(x_flat, w, b)


def _attn_one_head(q, k, v, c_ref, *, Lx, u):
    TS = 256
    ntile = Lx // TS

    cols = []
    for t in range(ntile):
        qt = q[t * TS:(t + 1) * TS, :]
        st = lax.dot_general(qt, k, (((1,), (1,)), ((), ())),
                             precision=PREC, preferred_element_type=F32)
        ct = c_ref[t * TS:(t + 1) * TS, :].astype(F32)
        ssum = jnp.sum(st * ct, axis=1, keepdims=True)
        smax = jnp.max(jnp.where(ct > 0, st, -jnp.inf), axis=1, keepdims=True)
        cols.append(smax - ssum / Lx)
    M = jnp.concatenate(cols, axis=1)
    m_row = jnp.reshape(jnp.transpose(M), (1, Lx))
    iota_lane = lax.broadcasted_iota(jnp.int32, (1, Lx), 1)
    BF = jnp.bfloat16
    ones_l = jnp.ones((Lx, 1), BF)
    ranks = []
    for t in range(ntile):
        m_col = cols[t]
        idx_col = (t * TS
                   + lax.broadcasted_iota(jnp.int32, (TS, 1), 0))
        cntf = ((m_row > m_col).astype(BF)
                + ((m_row == m_col) & (iota_lane < idx_col)).astype(BF))
        ranks.append(lax.dot_general(cntf, ones_l, (((1,), (0,)), ((), ())),
                                     preferred_element_type=F32))
    rank_col = jnp.concatenate(ranks, axis=0)

    U_PAD = 24
    slot = lax.broadcasted_iota(jnp.int32, (1, U_PAD), 1)
    onehot_t = ((rank_col == slot.astype(F32)) & (slot < u)).astype(F32)

    q_red = lax.dot_general(onehot_t, q, (((0,), (0,)), ((), ())),
                            precision=PREC, preferred_element_type=F32)
    scores = lax.dot_general(q_red, k, (((1,), (1,)), ((), ())),
                             precision=PREC, preferred_element_type=F32)
    scores = scores / jnp.sqrt(jnp.float32(DK))
    smax = jnp.max(scores, axis=1, keepdims=True)
    e = jnp.exp(scores - smax)
    attn = e / jnp.sum(e, axis=1, keepdims=True)
    upd = jnp.dot(attn, v, precision=PREC, preferred_element_type=F32)

    mean_v = jnp.mean(v, axis=0, keepdims=True)
    selcol = jnp.sum(onehot_t, axis=1, keepdims=True)
    scat = jnp.dot(onehot_t, upd, precision=PREC,
                   preferred_element_type=F32)
    return mean_v * (1.0 - selcol) + scat


def _attn_body(x_ref, wq_ref, wk_ref, wv_ref, bq_ref, bk_ref, bv_ref,
               c_ref, o_ref, *, Lx, u):
    x = x_ref[...]
    q2 = jnp.dot(x, wq_ref[...], precision=PREC,
                 preferred_element_type=F32) + bq_ref[...]
    k2 = jnp.dot(x, wk_ref[...], precision=PREC,
                 preferred_element_type=F32) + bk_ref[...]
    v2 = jnp.dot(x, wv_ref[...], precision=PREC,
                 preferred_element_type=F32) + bv_ref[...]
    parts = []
    for s in range(2):
        sl = slice(s * DK, (s + 1) * DK)
        parts.append(_attn_one_head(q2[:, sl], k2[:, sl], v2[:, sl],
                                    c_ref, Lx=Lx, u=u))
    o_ref[...] = jnp.concatenate(parts, axis=1)


def _attention(x_flat, lp, layer_idx, Lx):
    u, cmat = _sample_constants(layer_idx, Lx)
    body = functools.partial(_attn_body, Lx=Lx, u=u)
    HP = N_HEADS // 2
    wq = lp["Wq"].T
    wk = lp["Wk"].T
    wv = lp["Wv"].T
    w_spec = lambda off: pl.BlockSpec((D_MODEL, 2 * DK), lambda b, h: (0, h))
    b_spec = pl.BlockSpec((1, 2 * DK), lambda b, h: (0, h))
    return pl.pallas_call(
        body,
        grid=(B, HP),
        in_specs=[
            pl.BlockSpec((Lx, D_MODEL), lambda b, h: (b, 0)),
            w_spec(0), w_spec(0), w_spec(0),
            b_spec, b_spec, b_spec,
            pl.BlockSpec((Lx, Lx), lambda b, h: (0, 0)),
        ],
        out_specs=pl.BlockSpec((Lx, 2 * DK), lambda b, h: (b, h)),
        out_shape=jax.ShapeDtypeStruct((B * Lx, D_MODEL), F32),
    )(x_flat, wq, wk, wv, lp["bq"][None, :], lp["bk"][None, :],
      lp["bv"][None, :], cmat)


def _ln(x, g, b):
    m = jnp.mean(x, axis=-1, keepdims=True)
    v = jnp.mean((x - m) ** 2, axis=-1, keepdims=True)
    return (x - m) / jnp.sqrt(v + 1e-5) * g + b


def _dense_compute(xin_ref, ctx_ref, wo_ref, bo_ref, g1_ref, b1n_ref,
                   w1_ref, b1_ref, w2_ref, b2_ref, g2_ref, b2n_ref):
    a = jnp.dot(ctx_ref[...], wo_ref[...], precision=PREC,
                preferred_element_type=F32) + bo_ref[...]
    x = xin_ref[...] + a
    xn = _ln(x, g1_ref[...], b1n_ref[...])
    y = jnp.dot(xn, w1_ref[...], precision=PREC, preferred_element_type=F32) + b1_ref[...]
    y = y * 0.5 * (1.0 + lax.erf(y * (1.0 / np.sqrt(2.0).astype(np.float32))))
    z = jnp.dot(y, w2_ref[...], precision=PREC, preferred_element_type=F32) + b2_ref[...]
    return _ln(xn + z, g2_ref[...], b2n_ref[...])


def _dense_body(*refs):
    o_ref = refs[-1]
    o_ref[...] = _dense_compute(*refs[:-1])


def _dense_final_body(*refs, tiles_per_batch, Lx):
    o_ref = refs[-1]
    gf_ref, bf_ref = refs[-3], refs[-2]
    out = _dense_compute(*refs[:-3])
    xf = _ln(out, gf_ref[...], bf_ref[...])
    partial = (jnp.sum(xf, axis=0, keepdims=True) / Lx)[None]

    @pl.when(pl.program_id(0) % tiles_per_batch == 0)
    def _init():
        o_ref[...] = partial

    @pl.when(pl.program_id(0) % tiles_per_batch != 0)
    def _acc():
        o_ref[...] += partial


def _dense(x_flat, ctx_flat, lp, rows, final_gb=None):
    TM = 256
    nt = rows // TM
    wo = lp["Wo"].T
    w1 = lp["conv1_w"][:, :, 0].T
    w2 = lp["conv2_w"][:, :, 0].T
    vec = lambda a: a[None, :]
    in_specs = [
        pl.BlockSpec((TM, D_MODEL), lambda i: (i, 0)),
        pl.BlockSpec((TM, D_MODEL), lambda i: (i, 0)),
        pl.BlockSpec((D_MODEL, D_MODEL), lambda i: (0, 0)),
        pl.BlockSpec((1, D_MODEL), lambda i: (0, 0)),
        pl.BlockSpec((1, D_MODEL), lambda i: (0, 0)),
        pl.BlockSpec((1, D_MODEL), lambda i: (0, 0)),
        pl.BlockSpec((D_MODEL, D_FF), lambda i: (0, 0)),
        pl.BlockSpec((1, D_FF), lambda i: (0, 0)),
        pl.BlockSpec((D_FF, D_MODEL), lambda i: (0, 0)),
        pl.BlockSpec((1, D_MODEL), lambda i: (0, 0)),
        pl.BlockSpec((1, D_MODEL), lambda i: (0, 0)),
        pl.BlockSpec((1, D_MODEL), lambda i: (0, 0)),
    ]
    args = [x_flat, ctx_flat, wo, vec(lp["bo"]), vec(lp["n1_g"]),
            vec(lp["n1_b"]), w1, vec(lp["conv1_b"]), w2, vec(lp["conv2_b"]),
            vec(lp["n2_g"]), vec(lp["n2_b"])]
    if final_gb is None:
        return pl.pallas_call(
            _dense_body,
            grid=(nt,),
            in_specs=in_specs,
            out_specs=pl.BlockSpec((TM, D_MODEL), lambda i: (i, 0)),
            out_shape=jax.ShapeDtypeStruct((rows, D_MODEL), F32),
        )(*args)
    tpb = nt // B
    Lx = rows // B
    in_specs += [pl.BlockSpec((1, D_MODEL), lambda i: (0, 0))] * 2
    args += [vec(final_gb[0]), vec(final_gb[1])]
    out = pl.pallas_call(
        functools.partial(_dense_final_body, tiles_per_batch=tpb, Lx=Lx),
        grid=(nt,),
        in_specs=in_specs,
        out_specs=pl.BlockSpec((1, 1, D_MODEL), lambda i: (i // tpb, 0, 0)),
        out_shape=jax.ShapeDtypeStruct((B, 1, D_MODEL), F32),
    )(*args)
    return out.reshape(B, D_MODEL)


def _distill_body(xe_ref, xo_ref, xom1_ref, xem1_ref, xep1_ref,
                  w0_ref, w1_ref, w2_ref, a_ref, bt_ref, o_ref, *, TM):
    w0 = w0_ref[...]
    w1 = w1_ref[...]
    w2 = w2_ref[...]
    a = a_ref[...]
    bt = bt_ref[...]

    def conv(xa, xb, xc):
        h = (jnp.dot(xa, w0, precision=PREC, preferred_element_type=F32)
             + jnp.dot(xb, w1, precision=PREC, preferred_element_type=F32)
             + jnp.dot(xc, w2, precision=PREC, preferred_element_type=F32))
        h = h * a + bt
        return jnp.where(h > 0, h, jnp.exp(jnp.minimum(h, 0.0)) - 1.0)

    he = conv(xom1_ref[...], xe_ref[...], xo_ref[...])
    ho = conv(xe_ref[...], xo_ref[...], xep1_ref[...])
    hm = conv(xem1_ref[...], xom1_ref[...], xe_ref[...])
    grow = pl.program_id(1) * TM + lax.broadcasted_iota(jnp.int32, hm.shape, 0)
    hm = jnp.where(grow == 0, -jnp.inf, hm)
    o_ref[...] = jnp.maximum(jnp.maximum(hm, he), ho)


def _distill(x_flat, cp, Lx):
    L2 = Lx // 2
    x4 = x_flat.reshape(B, L2, 2, D_MODEL)
    xe = x4[:, :, 0, :]
    xo = x4[:, :, 1, :]
    xo_m1 = jnp.roll(xo, 1, axis=1)
    xe_m1 = jnp.roll(xe, 1, axis=1)
    xe_p1 = jnp.roll(xe, -1, axis=1)
    flat = lambda t: t.reshape(B * L2, D_MODEL)
    w0 = cp["dc_w"][:, :, 0].T
    w1 = cp["dc_w"][:, :, 1].T
    w2 = cp["dc_w"][:, :, 2].T
    a = (cp["bn_g"] / jnp.sqrt(1.0 + 1e-5))[None, :]
    bt = cp["dc_b"][None, :] * a + cp["bn_b"][None, :]
    TM = 128
    nt = L2 // TM
    row_spec = pl.BlockSpec((TM, D_MODEL), lambda b, t: (b * nt + t, 0))
    w_spec = pl.BlockSpec((D_MODEL, D_MODEL), lambda b, t: (0, 0))
    v_spec = pl.BlockSpec((1, D_MODEL), lambda b, t: (0, 0))
    return pl.pallas_call(
        functools.partial(_distill_body, TM=TM),
        grid=(B, nt),
        in_specs=[row_spec] * 5 + [w_spec] * 3 + [v_spec] * 2,
        out_specs=row_spec,
        out_shape=jax.ShapeDtypeStruct((B * L2, D_MODEL), F32),
    )(flat(xe), flat(xo), flat(xo_m1), flat(xe_m1), flat(xe_p1),
      w0, w1, w2, a, bt)


def _final_body(x_ref, g_ref, b_ref, o_ref):
    xn = _ln(x_ref[...], g_ref[...], b_ref[...])
    o_ref[...] = jnp.mean(xn, axis=0, keepdims=True)[None]


def _final(x_flat, g, b, Lx):
    out = pl.pallas_call(
        _final_body,
        grid=(B,),
        in_specs=[
            pl.BlockSpec((Lx, D_MODEL), lambda i: (i, 0)),
            pl.BlockSpec((1, D_MODEL), lambda i: (0, 0)),
            pl.BlockSpec((1, D_MODEL), lambda i: (0, 0)),
        ],
        out_specs=pl.BlockSpec((1, 1, D_MODEL), lambda i: (i, 0, 0)),
        out_shape=jax.ShapeDtypeStruct((B, 1, D_MODEL), F32),
    )(x_flat, g[None, :], b[None, :])
    return out.reshape(B, D_MODEL)


def kernel(x_enc, params):
    x = _embed(x_enc.reshape(B * L0, C_IN), params["tok_conv_w"])
    Lx = L0
    for l in range(E_LAYERS):
        rows = B * Lx
        ctx = _attention(x, params["layers"][l], l, Lx)
        if l < E_LAYERS - 1:
            x = _dense(x, ctx, params["layers"][l], rows)
            x = _distill(x, params["convs"][l], Lx)
            Lx //= 2
        else:
            x = _dense(x, ctx, params["layers"][l], rows,
                       final_gb=(params["norm_g"], params["norm_b"]))
    return x


# speedup vs baseline: 2.1061x; 1.0022x over previous
import functools
import math

import numpy as np
import jax
import jax.numpy as jnp
from jax import lax
from jax.experimental import pallas as pl
from jax.experimental.pallas import tpu as pltpu

B, L0, C_IN = 2, 2048, 7
D_MODEL, N_HEADS, E_LAYERS = 1024, 16, 3
D_FF, FACTOR = 512, 3
DK = D_MODEL // N_HEADS

PREC = lax.Precision.DEFAULT
F32 = jnp.float32

def _sample_constants_eager():
    consts = []
    base = jax.random.key(42)
    for l in range(E_LAYERS):
        Ll = L0 >> l
        u = min(int(FACTOR * np.ceil(np.log(Ll))), Ll)
        idx = np.asarray(jax.random.randint(jax.random.fold_in(base, l),
                                            (Ll, u), 0, Ll))
        cnt = np.zeros((Ll, Ll), np.float32)
        np.add.at(cnt, (np.arange(Ll)[:, None], idx), 1.0)
        consts.append((u, cnt.astype(np.int8)))
    return consts


with jax.default_device(jax.local_devices(backend="cpu")[0]):
    _SAMPLE_CONSTS = _sample_constants_eager()


def _sample_constants(layer_idx, Ll):
    u, cnt = _SAMPLE_CONSTS[layer_idx]
    return u, jnp.asarray(cnt)


def _embed_body(x_ref, w0_ref, w1_ref, w2_ref, o_ref):
    x = x_ref[...]
    xm1 = jnp.concatenate([x[-1:, :], x[:-1, :]], axis=0)
    xp1 = jnp.concatenate([x[1:, :], x[:1, :]], axis=0)
    o_ref[...] = (
        jnp.dot(xm1, w0_ref[...], precision=PREC, preferred_element_type=F32)
        + jnp.dot(x, w1_ref[...], precision=PREC, preferred_element_type=F32)
        + jnp.dot(xp1, w2_ref[...], precision=PREC, preferred_element_type=F32)
    )


def _embed(x_flat, w):
    w0 = w[:, :, 0].T
    w1 = w[:, :, 1].T
    w2 = w[:, :, 2].T
    return pl.pallas_call(
        _embed_body,
        grid=(B,),
        in_specs=[
            pl.BlockSpec((L0, C_IN), lambda b: (b, 0)),
            pl.BlockSpec((C_IN, D_MODEL), lambda b: (0, 0)),
            pl.BlockSpec((C_IN, D_MODEL), lambda b: (0, 0)),
            pl.BlockSpec((C_IN, D_MODEL), lambda b: (0, 0)),
        ],
        out_specs=pl.BlockSpec((L0, D_MODEL), lambda b: (b, 0)),
        out_shape=jax.ShapeDtypeStruct((B * L0, D_MODEL), F32),
    )(x_flat, w0, w1, w2)


def _attn_one_head(q, k, v, c_ref, *, Lx, u):
    TS = 256
    ntile = Lx // TS

    cols = []
    for t in range(ntile):
        qt = q[t * TS:(t + 1) * TS, :]
        st = lax.dot_general(qt, k, (((1,), (1,)), ((), ())),
                             precision=PREC, preferred_element_type=F32)
        ct = c_ref[t * TS:(t + 1) * TS, :].astype(F32)
        ssum = jnp.sum(st * ct, axis=1, keepdims=True)
        smax = jnp.max(jnp.where(ct > 0, st, -jnp.inf), axis=1, keepdims=True)
        cols.append(smax - ssum / Lx)
    M = jnp.concatenate(cols, axis=1)
    m_row = jnp.reshape(jnp.transpose(M), (1, Lx))
    iota_lane = lax.broadcasted_iota(jnp.int32, (1, Lx), 1)
    BF = jnp.bfloat16
    ones_l = jnp.ones((Lx, 1), BF)
    ranks = []
    for t in range(ntile):
        m_col = cols[t]
        idx_col = (t * TS
                   + lax.broadcasted_iota(jnp.int32, (TS, 1), 0))
        cntf = ((m_row > m_col).astype(BF)
                + ((m_row == m_col) & (iota_lane < idx_col)).astype(BF))
        ranks.append(lax.dot_general(cntf, ones_l, (((1,), (0,)), ((), ())),
                                     preferred_element_type=F32))
    rank_col = jnp.concatenate(ranks, axis=0)

    U_PAD = 24
    slot = lax.broadcasted_iota(jnp.int32, (1, U_PAD), 1)
    onehot_t = ((rank_col == slot.astype(F32)) & (slot < u)).astype(F32)

    q_red = lax.dot_general(onehot_t, q, (((0,), (0,)), ((), ())),
                            precision=PREC, preferred_element_type=F32)
    scores = lax.dot_general(q_red, k, (((1,), (1,)), ((), ())),
                             precision=PREC, preferred_element_type=F32)
    scores = scores / jnp.sqrt(jnp.float32(DK))
    smax = jnp.max(scores, axis=1, keepdims=True)
    e = jnp.exp(scores - smax)
    attn = e / jnp.sum(e, axis=1, keepdims=True)
    upd = jnp.dot(attn, v, precision=PREC, preferred_element_type=F32)

    mean_v = jnp.mean(v, axis=0, keepdims=True)
    selcol = jnp.sum(onehot_t, axis=1, keepdims=True)
    scat = jnp.dot(onehot_t, upd, precision=PREC,
                   preferred_element_type=F32)
    return mean_v * (1.0 - selcol) + scat


def _attn_body(x_ref, wq_ref, wk_ref, wv_ref, bq_ref, bk_ref, bv_ref,
               c_ref, o_ref, *, Lx, u):
    x = x_ref[...]
    q2 = jnp.dot(x, wq_ref[...], precision=PREC,
                 preferred_element_type=F32) + bq_ref[...]
    k2 = jnp.dot(x, wk_ref[...], precision=PREC,
                 preferred_element_type=F32) + bk_ref[...]
    v2 = jnp.dot(x, wv_ref[...], precision=PREC,
                 preferred_element_type=F32) + bv_ref[...]
    parts = []
    for s in range(2):
        sl = slice(s * DK, (s + 1) * DK)
        parts.append(_attn_one_head(q2[:, sl], k2[:, sl], v2[:, sl],
                                    c_ref, Lx=Lx, u=u))
    o_ref[...] = jnp.concatenate(parts, axis=1)


def _attention(x_flat, lp, layer_idx, Lx):
    u, cmat = _sample_constants(layer_idx, Lx)
    body = functools.partial(_attn_body, Lx=Lx, u=u)
    HP = N_HEADS // 2
    wq = lp["Wq"].T
    wk = lp["Wk"].T
    wv = lp["Wv"].T
    w_spec = lambda off: pl.BlockSpec((D_MODEL, 2 * DK), lambda b, h: (0, h))
    b_spec = pl.BlockSpec((1, 2 * DK), lambda b, h: (0, h))
    return pl.pallas_call(
        body,
        grid=(B, HP),
        in_specs=[
            pl.BlockSpec((Lx, D_MODEL), lambda b, h: (b, 0)),
            w_spec(0), w_spec(0), w_spec(0),
            b_spec, b_spec, b_spec,
            pl.BlockSpec((Lx, Lx), lambda b, h: (0, 0)),
        ],
        out_specs=pl.BlockSpec((Lx, 2 * DK), lambda b, h: (b, h)),
        out_shape=jax.ShapeDtypeStruct((B * Lx, D_MODEL), F32),
    )(x_flat, wq, wk, wv, lp["bq"][None, :], lp["bk"][None, :],
      lp["bv"][None, :], cmat)


def _ln(x, g, b):
    m = jnp.mean(x, axis=-1, keepdims=True)
    v = jnp.mean((x - m) ** 2, axis=-1, keepdims=True)
    return (x - m) / jnp.sqrt(v + 1e-5) * g + b


def _dense_compute(xin_ref, ctx_ref, wo_ref, bo_ref, g1_ref, b1n_ref,
                   w1_ref, b1_ref, w2_ref, b2_ref, g2_ref, b2n_ref):
    a = jnp.dot(ctx_ref[...], wo_ref[...], precision=PREC,
                preferred_element_type=F32) + bo_ref[...]
    x = xin_ref[...] + a
    xn = _ln(x, g1_ref[...], b1n_ref[...])
    y = jnp.dot(xn, w1_ref[...], precision=PREC, preferred_element_type=F32) + b1_ref[...]
    y = y * 0.5 * (1.0 + lax.erf(y * (1.0 / np.sqrt(2.0).astype(np.float32))))
    z = jnp.dot(y, w2_ref[...], precision=PREC, preferred_element_type=F32) + b2_ref[...]
    return _ln(xn + z, g2_ref[...], b2n_ref[...])


def _dense_body(*refs):
    o_ref = refs[-1]
    o_ref[...] = _dense_compute(*refs[:-1])


def _dense_final_body(*refs, tiles_per_batch, Lx):
    o_ref = refs[-1]
    gf_ref, bf_ref = refs[-3], refs[-2]
    out = _dense_compute(*refs[:-3])
    xf = _ln(out, gf_ref[...], bf_ref[...])
    partial = (jnp.sum(xf, axis=0, keepdims=True) / Lx)[None]

    @pl.when(pl.program_id(0) % tiles_per_batch == 0)
    def _init():
        o_ref[...] = partial

    @pl.when(pl.program_id(0) % tiles_per_batch != 0)
    def _acc():
        o_ref[...] += partial


def _dense(x_flat, ctx_flat, lp, rows, final_gb=None):
    TM = 256
    nt = rows // TM
    wo = lp["Wo"].T
    w1 = lp["conv1_w"][:, :, 0].T
    w2 = lp["conv2_w"][:, :, 0].T
    vec = lambda a: a[None, :]
    in_specs = [
        pl.BlockSpec((TM, D_MODEL), lambda i: (i, 0)),
        pl.BlockSpec((TM, D_MODEL), lambda i: (i, 0)),
        pl.BlockSpec((D_MODEL, D_MODEL), lambda i: (0, 0)),
        pl.BlockSpec((1, D_MODEL), lambda i: (0, 0)),
        pl.BlockSpec((1, D_MODEL), lambda i: (0, 0)),
        pl.BlockSpec((1, D_MODEL), lambda i: (0, 0)),
        pl.BlockSpec((D_MODEL, D_FF), lambda i: (0, 0)),
        pl.BlockSpec((1, D_FF), lambda i: (0, 0)),
        pl.BlockSpec((D_FF, D_MODEL), lambda i: (0, 0)),
        pl.BlockSpec((1, D_MODEL), lambda i: (0, 0)),
        pl.BlockSpec((1, D_MODEL), lambda i: (0, 0)),
        pl.BlockSpec((1, D_MODEL), lambda i: (0, 0)),
    ]
    args = [x_flat, ctx_flat, wo, vec(lp["bo"]), vec(lp["n1_g"]),
            vec(lp["n1_b"]), w1, vec(lp["conv1_b"]), w2, vec(lp["conv2_b"]),
            vec(lp["n2_g"]), vec(lp["n2_b"])]
    if final_gb is None:
        return pl.pallas_call(
            _dense_body,
            grid=(nt,),
            in_specs=in_specs,
            out_specs=pl.BlockSpec((TM, D_MODEL), lambda i: (i, 0)),
            out_shape=jax.ShapeDtypeStruct((rows, D_MODEL), F32),
        )(*args)
    tpb = nt // B
    Lx = rows // B
    in_specs += [pl.BlockSpec((1, D_MODEL), lambda i: (0, 0))] * 2
    args += [vec(final_gb[0]), vec(final_gb[1])]
    out = pl.pallas_call(
        functools.partial(_dense_final_body, tiles_per_batch=tpb, Lx=Lx),
        grid=(nt,),
        in_specs=in_specs,
        out_specs=pl.BlockSpec((1, 1, D_MODEL), lambda i: (i // tpb, 0, 0)),
        out_shape=jax.ShapeDtypeStruct((B, 1, D_MODEL), F32),
    )(*args)
    return out.reshape(B, D_MODEL)


def _distill_body(xe_ref, xo_ref, xom1_ref, xem1_ref, xep1_ref,
                  w0_ref, w1_ref, w2_ref, a_ref, bt_ref, o_ref, *, TM):
    w0 = w0_ref[...]
    w1 = w1_ref[...]
    w2 = w2_ref[...]
    a = a_ref[...]
    bt = bt_ref[...]

    def conv(xa, xb, xc):
        h = (jnp.dot(xa, w0, precision=PREC, preferred_element_type=F32)
             + jnp.dot(xb, w1, precision=PREC, preferred_element_type=F32)
             + jnp.dot(xc, w2, precision=PREC, preferred_element_type=F32))
        h = h * a + bt
        return jnp.where(h > 0, h, jnp.exp(jnp.minimum(h, 0.0)) - 1.0)

    he = conv(xom1_ref[...], xe_ref[...], xo_ref[...])
    ho = conv(xe_ref[...], xo_ref[...], xep1_ref[...])
    hm = conv(xem1_ref[...], xom1_ref[...], xe_ref[...])
    grow = pl.program_id(1) * TM + lax.broadcasted_iota(jnp.int32, hm.shape, 0)
    hm = jnp.where(grow == 0, -jnp.inf, hm)
    o_ref[...] = jnp.maximum(jnp.maximum(hm, he), ho)


def _distill(x_flat, cp, Lx):
    L2 = Lx // 2
    x4 = x_flat.reshape(B, L2, 2, D_MODEL)
    xe = x4[:, :, 0, :]
    xo = x4[:, :, 1, :]
    xo_m1 = jnp.roll(xo, 1, axis=1)
    xe_m1 = jnp.roll(xe, 1, axis=1)
    xe_p1 = jnp.roll(xe, -1, axis=1)
    flat = lambda t: t.reshape(B * L2, D_MODEL)
    w0 = cp["dc_w"][:, :, 0].T
    w1 = cp["dc_w"][:, :, 1].T
    w2 = cp["dc_w"][:, :, 2].T
    a = (cp["bn_g"] / jnp.sqrt(1.0 + 1e-5))[None, :]
    bt = cp["dc_b"][None, :] * a + cp["bn_b"][None, :]
    TM = 128
    nt = L2 // TM
    row_spec = pl.BlockSpec((TM, D_MODEL), lambda b, t: (b * nt + t, 0))
    w_spec = pl.BlockSpec((D_MODEL, D_MODEL), lambda b, t: (0, 0))
    v_spec = pl.BlockSpec((1, D_MODEL), lambda b, t: (0, 0))
    return pl.pallas_call(
        functools.partial(_distill_body, TM=TM),
        grid=(B, nt),
        in_specs=[row_spec] * 5 + [w_spec] * 3 + [v_spec] * 2,
        out_specs=row_spec,
        out_shape=jax.ShapeDtypeStruct((B * L2, D_MODEL), F32),
    )(flat(xe), flat(xo), flat(xo_m1), flat(xe_m1), flat(xe_p1),
      w0, w1, w2, a, bt)


def kernel(x_enc, params):
    x = _embed(x_enc.reshape(B * L0, C_IN), params["tok_conv_w"])
    Lx = L0
    for l in range(E_LAYERS):
        rows = B * Lx
        ctx = _attention(x, params["layers"][l], l, Lx)
        if l < E_LAYERS - 1:
            x = _dense(x, ctx, params["layers"][l], rows)
            x = _distill(x, params["convs"][l], Lx)
            Lx //= 2
        else:
            x = _dense(x, ctx, params["layers"][l], rows,
                       final_gb=(params["norm_g"], params["norm_b"]))
    return x
